# Initial kernel scaffold; baseline (speedup 1.0000x reference)
import jax
import jax.numpy as jnp
from jax.experimental import pallas as pl


def kernel(x, embedding):
    raise NotImplementedError("write your pallas kernel here")


# speedup vs baseline: 1.8139x; 1.8139x over previous
import jax
import jax.numpy as jnp
from jax import lax
from jax.experimental import pallas as pl
from jax.experimental.pallas import tpu as pltpu
from jax.experimental.pallas import tpu_sc as plsc

_NC = 2
_NS = 16
_NW = _NC * _NS

_SUB = 128
_NSUB = 5
_CHUNK = _SUB * _NSUB


def _make_gather(vocab, dim, n):
    assert n % (_NW * _CHUNK) == 0
    b_per_w = n // _NW
    n_chunk = b_per_w // _CHUNK

    mesh = plsc.VectorSubcoreMesh(core_axis_name="c", subcore_axis_name="s")

    @pl.kernel(
        out_type=jax.ShapeDtypeStruct((n, dim), jnp.float32),
        mesh=mesh,
        scratch_types=[
            pltpu.VMEM((_NSUB, _SUB), jnp.int32),
            pltpu.VMEM((_CHUNK, dim), jnp.float32),
            pltpu.SemaphoreType.DMA,
        ],
        compiler_params=pltpu.CompilerParams(use_tc_tiling_on_sc=False),
    )
    def gather_kernel(idx_hbm, table_hbm, out_hbm, idx_v, rows_v, sem):
        wid = lax.axis_index("s") * _NC + lax.axis_index("c")

        def chunk_body(i, carry):
            pltpu.sync_copy(idx_hbm.at[wid, i], idx_v)
            handles = [
                pltpu.async_copy(
                    table_hbm.at[idx_v.at[j]],
                    rows_v.at[pl.ds(j * _SUB, _SUB)],
                    sem,
                )
                for j in range(_NSUB)
            ]
            for h in handles:
                h.wait()
            pltpu.sync_copy(
                rows_v, out_hbm.at[pl.ds(wid * b_per_w + i * _CHUNK, _CHUNK)]
            )
            return carry

        lax.fori_loop(0, n_chunk, chunk_body, 0)

    return gather_kernel


def kernel(x, embedding):
    batch, hist = x.shape
    vocab, dim = embedding.shape
    n = batch * hist
    idx = x.reshape(_NW, n // (_NW * _NSUB * _SUB), _NSUB, _SUB)
    out = _make_gather(vocab, dim, n)(idx, embedding)
    return out.reshape(batch, hist, dim)


# speedup vs baseline: 1.8765x; 1.0345x over previous
import jax
import jax.numpy as jnp
from jax import lax
from jax.experimental import pallas as pl
from jax.experimental.pallas import tpu as pltpu
from jax.experimental.pallas import tpu_sc as plsc

_NC = 2
_NS = 16
_NW = _NC * _NS

_SUB = 128
_NSUB = 4
_CHUNK = _SUB * _NSUB


def _make_gather(vocab, dim, n):
    assert n % (_NW * 2 * _CHUNK) == 0
    b_per_w = n // _NW
    n_chunk = b_per_w // _CHUNK
    n_sub_total = b_per_w // _SUB

    mesh = plsc.VectorSubcoreMesh(core_axis_name="c", subcore_axis_name="s")

    @pl.kernel(
        out_type=jax.ShapeDtypeStruct((n, dim), jnp.float32),
        mesh=mesh,
        scratch_types=[
            pltpu.VMEM((n_sub_total, _SUB), jnp.int32),
            pltpu.VMEM((2, _CHUNK, dim), jnp.float32),
            pltpu.SemaphoreType.DMA,
            pltpu.SemaphoreType.DMA,
            pltpu.SemaphoreType.DMA,
            pltpu.SemaphoreType.DMA,
        ],
        compiler_params=pltpu.CompilerParams(use_tc_tiling_on_sc=False),
    )
    def gather_kernel(idx_hbm, table_hbm, out_hbm, idx_v, rows_v, g0, g1, o0, o1):
        wid = lax.axis_index("s") * _NC + lax.axis_index("c")
        base = wid * b_per_w
        sem_g = (g0, g1)
        sem_o = (o0, o1)

        def start_gathers(i, b):
            for j in range(_NSUB):
                pltpu.async_copy(
                    table_hbm.at[idx_v.at[i * _NSUB + j]],
                    rows_v.at[b, pl.ds(j * _SUB, _SUB)],
                    sem_g[b],
                )

        def wait_gathers(i, b):
            for j in range(_NSUB):
                pltpu.make_async_copy(
                    table_hbm.at[idx_v.at[i * _NSUB + j]],
                    rows_v.at[b, pl.ds(j * _SUB, _SUB)],
                    sem_g[b],
                ).wait()

        def start_out(i, b):
            pltpu.async_copy(
                rows_v.at[b], out_hbm.at[pl.ds(base + i * _CHUNK, _CHUNK)], sem_o[b]
            )

        def wait_out(i, b):
            pltpu.make_async_copy(
                rows_v.at[b], out_hbm.at[pl.ds(base + i * _CHUNK, _CHUNK)], sem_o[b]
            ).wait()

        pltpu.sync_copy(idx_hbm.at[wid], idx_v)

        start_gathers(0, 0)
        start_gathers(1, 1)
        wait_gathers(0, 0)
        start_out(0, 0)

        def pair_body(k, carry):
            i0 = 2 + 2 * k
            for di in range(2):
                i = i0 + di
                b = di
                wait_out(i - 2, b)
                start_gathers(i, b)
                wait_gathers(i - 1, 1 - b)
                start_out(i - 1, 1 - b)
            return carry

        lax.fori_loop(0, (n_chunk - 2) // 2, pair_body, 0)

        wait_gathers(n_chunk - 1, 1)
        start_out(n_chunk - 1, 1)
        wait_out(n_chunk - 2, 0)
        wait_out(n_chunk - 1, 1)

    return gather_kernel


def kernel(x, embedding):
    batch, hist = x.shape
    vocab, dim = embedding.shape
    n = batch * hist
    idx = x.reshape(_NW, n // (_NW * _SUB), _SUB)
    out = _make_gather(vocab, dim, n)(idx, embedding)
    return out.reshape(batch, hist, dim)
